# baseline (speedup 1.0000x reference)
import functools

import jax
import jax.numpy as jnp
import numpy as np
from jax import lax
from jax.experimental import pallas as pl
from jax.experimental.pallas import tpu as pltpu
from jax.experimental.pallas import tpu_sc as plsc

_B = 128
_NODES_PER = 336
_N = _B * _NODES_PER
_D = 64
_HD = _D // 2
_E = 688128
_NCLS = 10

_NC = 2
_NS = 16
_L = 16

_G = 128
_NGROUPS = _E // _G
_ROWS_PER_TILE = _N // _NS


def _pe_rows(seq_len, d, n=10000.0):
    k = np.arange(seq_len, dtype=np.float32)[:, None]
    i = np.arange(d // 2, dtype=np.float32)[None, :]
    denom = np.power(np.float32(n), 2.0 * i / np.float32(d))
    P = np.zeros((seq_len, d), np.float32)
    P[:, 0::2] = np.sin(k / denom)
    P[:, 1::2] = np.cos(k / denom)
    return P


def _build_pe():
    pe = np.zeros((_NODES_PER, _D), np.float32)
    f4 = _pe_rows(4, _D)
    f16 = _pe_rows(16, _D)
    for i in range(4):
        pe[16 + 16 * i:32 + 16 * i] = f4[i]
    for i in range(16):
        pe[80 + 16 * i:96 + 16 * i] = f16[i]
    return pe


_PE8_NP = np.tile(_build_pe(), (8, 1))


_H_GPT = _NGROUPS // (_NC * _NS)
_H_CHUNK = 24


_H_EPT = _E // (_NC * _NS)
_H_ECHUNK = _H_CHUNK * _G


def _hist_body(col_hbm, deg_out, colb, degl):
    c = lax.axis_index("c")
    s = lax.axis_index("s")
    wid = c * _NS + s
    z16 = jnp.zeros((_L,), jnp.float32)
    ones16 = jnp.ones((_L,), jnp.float32)

    @pl.loop(0, _N // _L)
    def _zero(i):
        degl[pl.ds(i * _L, _L)] = z16

    e0 = wid * _H_EPT

    @pl.loop(0, _H_EPT // _H_ECHUNK)
    def _chunk(b):
        pltpu.sync_copy(col_hbm.at[pl.ds(e0 + b * _H_ECHUNK, _H_ECHUNK)],
                        colb)

        @pl.loop(0, _H_ECHUNK // _L)
        def _vec(k):
            idx = colb[pl.ds(k * _L, _L)]
            plsc.addupdate_scatter(degl, [idx], ones16)

    pltpu.sync_copy(degl, deg_out.at[wid])


def _hist(col1d):
    mesh = plsc.VectorSubcoreMesh(core_axis_name="c", subcore_axis_name="s")
    return pl.kernel(
        _hist_body,
        out_type=jax.ShapeDtypeStruct((_NC * _NS, _N), jnp.float32),
        mesh=mesh,
        scratch_types=[
            pltpu.VMEM((_H_ECHUNK,), jnp.int32),
            pltpu.VMEM((_N,), jnp.float32),
        ],
        compiler_params=pltpu.CompilerParams(needs_layout_passes=False),
    )(col1d)


_PREP_BLK = 2688
_PREP_GRID = _N // _PREP_BLK
_FBLK = _PREP_BLK // 4


def _prepa_body(x_ref, pos_ref, pe_ref, w_ref, xt0_ref, xt1_ref):
    h = (x_ref[...].reshape(_PREP_BLK, _D) +
         pos_ref[...].reshape(_PREP_BLK, _D) + pe_ref[...])
    xt = jnp.dot(h, w_ref[...], preferred_element_type=jnp.float32)
    xt0_ref[...] = xt[:, :_HD]
    xt1_ref[...] = xt[:, _HD:]


def _prep_a(xr, pos, pe8, w):
    return pl.pallas_call(
        _prepa_body,
        grid=(_PREP_GRID,),
        in_specs=[
            pl.BlockSpec((8, _NODES_PER, _D), lambda i: (i, 0, 0)),
            pl.BlockSpec((8, _NODES_PER, _D), lambda i: (i, 0, 0)),
            pl.BlockSpec((_PREP_BLK, _D), lambda i: (0, 0)),
            pl.BlockSpec((_D, _D), lambda i: (0, 0)),
        ],
        out_specs=[
            pl.BlockSpec((_PREP_BLK, _HD), lambda i: (i, 0)),
            pl.BlockSpec((_PREP_BLK, _HD), lambda i: (i, 0)),
        ],
        out_shape=[
            jax.ShapeDtypeStruct((_N, _HD), jnp.float32),
            jax.ShapeDtypeStruct((_N, _HD), jnp.float32),
        ],
    )(xr, pos, pe8, w)


_B_NODES = 7168
_B_GRID = _N // _B_NODES
_B_DROWS = _B_NODES // _G
_B_FROWS = _B_NODES // 4


def _prepb_body(xt0_ref, xt1_ref, degp_ref, y0_ref, y1_ref, dinv_ref):
    deg = jnp.sum(degp_ref[...], axis=0) + 1.0
    dinv_n = lax.rsqrt(deg)[:, None]
    y0_ref[...] = xt0_ref[...] * dinv_n
    y1_ref[...] = xt1_ref[...] * dinv_n
    dinv_ref[...] = dinv_n


def _prep_b(xt0, xt1, degp):
    return pl.pallas_call(
        _prepb_body,
        grid=(_B_GRID,),
        in_specs=[
            pl.BlockSpec((_B_NODES, _HD), lambda i: (i, 0)),
            pl.BlockSpec((_B_NODES, _HD), lambda i: (i, 0)),
            pl.BlockSpec((_NC * _NS, _B_NODES), lambda i: (0, i)),
        ],
        out_specs=[
            pl.BlockSpec((_B_NODES, _HD), lambda i: (i, 0)),
            pl.BlockSpec((_B_NODES, _HD), lambda i: (i, 0)),
            pl.BlockSpec((_B_NODES, 1), lambda i: (i, 0)),
        ],
        out_shape=[
            jax.ShapeDtypeStruct((_N, _HD), jnp.float32),
            jax.ShapeDtypeStruct((_N, _HD), jnp.float32),
            jax.ShapeDtypeStruct((_N, 1), jnp.float32),
        ],
    )(xt0, xt1, degp)


_E_GPT = _NGROUPS // _NS
_E_CHUNK = 24
_E_SB = 8
_E_LA = 4


def _fire_gather(c, y0, y1, rowb, rbuf, gsem, j):
    src_idx = rowb.at[pl.ds(j * _G, _G)]

    @pl.when(c == 0)
    def _():
        pltpu.make_async_copy(y0.at[src_idx], rbuf.at[j % _E_SB],
                              gsem).start()

    @pl.when(c == 1)
    def _():
        pltpu.make_async_copy(y1.at[src_idx], rbuf.at[j % _E_SB],
                              gsem).start()


def _edge_body(y0f, y1f, row1d, col1d, a0f, a1f, rowb, colb, rbuf,
               gsem, ssem, a_sh):
    y0, y1, a0_out, a1_out = y0f, y1f, a0f, a1f
    c = lax.axis_index("c")
    s = lax.axis_index("s")
    z16 = jnp.zeros((_L,), jnp.float32)

    @pl.loop(0, _G)
    def _zrow(r):
        rbuf[0, r, pl.ds(0, _L)] = z16
        rbuf[0, r, pl.ds(_L, _L)] = z16

    r0 = s * _ROWS_PER_TILE

    @pl.loop(0, _ROWS_PER_TILE // _G)
    def _zcopy(i):
        pltpu.sync_copy(rbuf.at[0], a_sh.at[pl.ds(r0 + i * _G, _G)])

    plsc.subcore_barrier()

    g0 = s * _E_GPT

    @pl.loop(0, _E_GPT // _E_CHUNK)
    def _chunk(b):
        eb = (g0 + b * _E_CHUNK) * _G
        pltpu.sync_copy(row1d.at[pl.ds(eb, _E_CHUNK * _G)], rowb)
        pltpu.sync_copy(col1d.at[pl.ds(eb, _E_CHUNK * _G)], colb)

        for j in range(_E_LA):
            _fire_gather(c, y0, y1, rowb, rbuf, gsem, j)

        @pl.loop(0, _E_CHUNK)
        def _pipe(j):
            pltpu.make_async_copy(
                y0.at[rowb.at[pl.ds(j * _G, _G)]], rbuf.at[j % _E_SB],
                gsem).wait()
            pltpu.async_copy(
                rbuf.at[j % _E_SB],
                a_sh.at[colb.at[pl.ds(j * _G, _G)]],
                ssem, add=True)

            @pl.when(j >= _E_LA)
            def _():
                jd = j - _E_LA
                pltpu.make_async_copy(
                    rbuf.at[jd % _E_SB],
                    a_sh.at[colb.at[pl.ds(jd * _G, _G)]],
                    ssem).wait()

            @pl.when(j + _E_LA < _E_CHUNK)
            def _():
                _fire_gather(c, y0, y1, rowb, rbuf, gsem, j + _E_LA)

        for j in range(_E_CHUNK - _E_LA, _E_CHUNK):
            pltpu.make_async_copy(
                rbuf.at[j % _E_SB],
                a_sh.at[colb.at[pl.ds(j * _G, _G)]],
                ssem).wait()

    plsc.subcore_barrier()

    @pl.when(c == 0)
    def _():
        pltpu.sync_copy(a_sh.at[pl.ds(r0, _ROWS_PER_TILE)],
                        a0_out.at[pl.ds(r0, _ROWS_PER_TILE)])

    @pl.when(c == 1)
    def _():
        pltpu.sync_copy(a_sh.at[pl.ds(r0, _ROWS_PER_TILE)],
                        a1_out.at[pl.ds(r0, _ROWS_PER_TILE)])


def _edge(y0, y1, row1d, col1d):
    mesh = plsc.VectorSubcoreMesh(core_axis_name="c", subcore_axis_name="s")
    return pl.kernel(
        _edge_body,
        out_type=[
            jax.ShapeDtypeStruct((_N, _HD), jnp.float32),
            jax.ShapeDtypeStruct((_N, _HD), jnp.float32),
        ],
        mesh=mesh,
        scratch_types=[
            pltpu.VMEM((_E_CHUNK * _G,), jnp.int32),
            pltpu.VMEM((_E_CHUNK * _G,), jnp.int32),
            pltpu.VMEM((_E_SB, _G, _HD), jnp.float32),
            pltpu.SemaphoreType.DMA,
            pltpu.SemaphoreType.DMA,
            pltpu.VMEM_SHARED((_N, _HD), jnp.float32),
        ],
        compiler_params=pltpu.CompilerParams(needs_layout_passes=False,
                                             use_tc_tiling_on_sc=False),
    )(y0, y1, row1d, col1d)


_FIN_BLK = 2688
_FIN_GRID = _N // _FIN_BLK


def _fin_body(a0_ref, a1_ref, y0_ref, y1_ref, dinv_ref, batch_ref, b1_ref,
              lw_ref, lb_ref, out_ref, pooled, cnt):
    i = pl.program_id(0)

    @pl.when(i == 0)
    def _():
        pooled[...] = jnp.zeros_like(pooled)
        cnt[...] = jnp.zeros_like(cnt)

    A = jnp.concatenate([a0_ref[...], a1_ref[...]], axis=1)
    y = jnp.concatenate([y0_ref[...], y1_ref[...]], axis=1)
    h = jnp.maximum(dinv_ref[...] * (A + y) + b1_ref[...], 0.0)
    seg = jax.lax.broadcasted_iota(jnp.int32, (_FIN_BLK, _B), 1)
    m = (batch_ref[...] == seg).astype(jnp.float32)
    pooled[...] += lax.dot_general(m, h, (((0,), (0,)), ((), ())),
                                   preferred_element_type=jnp.float32)
    cnt[...] += jnp.sum(m, axis=0, keepdims=True)

    @pl.when(i == _FIN_GRID - 1)
    def _():
        c = jnp.maximum(cnt[...], 1.0)
        pm = pooled[...] / jnp.transpose(c)
        logits = jnp.dot(pm, lw_ref[...],
                         preferred_element_type=jnp.float32) + lb_ref[...]
        z = logits - jnp.max(logits, axis=1, keepdims=True)
        e = jnp.exp(z)
        out_ref[...] = e / jnp.sum(e, axis=1, keepdims=True)


def _finish(a0, a1, y0, y1, dinv, batch2d, b1, lw, lb):
    return pl.pallas_call(
        _fin_body,
        grid=(_FIN_GRID,),
        in_specs=[
            pl.BlockSpec((_FIN_BLK, _HD), lambda i: (i, 0)),
            pl.BlockSpec((_FIN_BLK, _HD), lambda i: (i, 0)),
            pl.BlockSpec((_FIN_BLK, _HD), lambda i: (i, 0)),
            pl.BlockSpec((_FIN_BLK, _HD), lambda i: (i, 0)),
            pl.BlockSpec((_FIN_BLK, 1), lambda i: (i, 0)),
            pl.BlockSpec((_FIN_BLK, 1), lambda i: (i, 0)),
            pl.BlockSpec((1, _D), lambda i: (0, 0)),
            pl.BlockSpec((_D, _NCLS), lambda i: (0, 0)),
            pl.BlockSpec((1, _NCLS), lambda i: (0, 0)),
        ],
        out_specs=pl.BlockSpec((_B, _NCLS), lambda i: (0, 0)),
        out_shape=jax.ShapeDtypeStruct((_B, _NCLS), jnp.float32),
        scratch_shapes=[
            pltpu.VMEM((_B, _D), jnp.float32),
            pltpu.VMEM((1, _B), jnp.float32),
        ],
    )(a0, a1, y0, y1, dinv, batch2d, b1, lw, lb)


def kernel(x, edge_index, batch, batch_size, pos_embed, conv1_w, conv1_b,
           lin_w, lin_b):
    del batch_size
    row1d = edge_index[0]
    col1d = edge_index[1]
    pe8 = jnp.asarray(_PE8_NP)

    degp = _hist(col1d)
    xt0, xt1 = _prep_a(x, pos_embed, pe8, conv1_w)
    y0, y1, dinv = _prep_b(xt0, xt1, degp)
    a0, a1 = _edge(y0, y1, row1d, col1d)
    return _finish(a0, a1, y0, y1, dinv, batch.reshape(_N, 1),
                   conv1_b.reshape(1, _D), lin_w, lin_b.reshape(1, _NCLS))


# speedup vs baseline: 1.0555x; 1.0555x over previous
import functools

import jax
import jax.numpy as jnp
import numpy as np
from jax import lax
from jax.experimental import pallas as pl
from jax.experimental.pallas import tpu as pltpu
from jax.experimental.pallas import tpu_sc as plsc

_B = 128
_NODES_PER = 336
_N = _B * _NODES_PER
_D = 64
_HD = _D // 2
_E = 688128
_NCLS = 10

_NC = 2
_NS = 16
_L = 16

_G = 128
_NGROUPS = _E // _G
_ROWS_PER_TILE = _N // _NS


def _pe_rows(seq_len, d, n=10000.0):
    k = np.arange(seq_len, dtype=np.float32)[:, None]
    i = np.arange(d // 2, dtype=np.float32)[None, :]
    denom = np.power(np.float32(n), 2.0 * i / np.float32(d))
    P = np.zeros((seq_len, d), np.float32)
    P[:, 0::2] = np.sin(k / denom)
    P[:, 1::2] = np.cos(k / denom)
    return P


def _build_pe():
    pe = np.zeros((_NODES_PER, _D), np.float32)
    f4 = _pe_rows(4, _D)
    f16 = _pe_rows(16, _D)
    for i in range(4):
        pe[16 + 16 * i:32 + 16 * i] = f4[i]
    for i in range(16):
        pe[80 + 16 * i:96 + 16 * i] = f16[i]
    return pe


_PE8_NP = np.tile(_build_pe(), (8, 1))


_H_GPT = _NGROUPS // (_NC * _NS)
_H_CHUNK = 24


_H_EPT = _E // (_NC * _NS)
_H_ECHUNK = _H_CHUNK * _G


def _hist_body(col_hbm, deg_out, colb, degl):
    c = lax.axis_index("c")
    s = lax.axis_index("s")
    wid = c * _NS + s
    z16 = jnp.zeros((_L,), jnp.float32)
    ones16 = jnp.ones((_L,), jnp.float32)

    @pl.loop(0, _N // _L)
    def _zero(i):
        degl[pl.ds(i * _L, _L)] = z16

    e0 = wid * _H_EPT

    @pl.loop(0, _H_EPT // _H_ECHUNK)
    def _chunk(b):
        pltpu.sync_copy(col_hbm.at[pl.ds(e0 + b * _H_ECHUNK, _H_ECHUNK)],
                        colb)

        @pl.loop(0, _H_ECHUNK // _L)
        def _vec(k):
            idx = colb[pl.ds(k * _L, _L)]
            plsc.addupdate_scatter(degl, [idx], ones16)

    pltpu.sync_copy(degl, deg_out.at[wid])


def _hist(col1d):
    mesh = plsc.VectorSubcoreMesh(core_axis_name="c", subcore_axis_name="s")
    return pl.kernel(
        _hist_body,
        out_type=jax.ShapeDtypeStruct((_NC * _NS, _N), jnp.float32),
        mesh=mesh,
        scratch_types=[
            pltpu.VMEM((_H_ECHUNK,), jnp.int32),
            pltpu.VMEM((_N,), jnp.float32),
        ],
        compiler_params=pltpu.CompilerParams(needs_layout_passes=False),
    )(col1d)


_PREP_BLK = 2688
_PREP_GRID = _N // _PREP_BLK
_FBLK = _PREP_BLK // 4


def _prepa_body(x_ref, pos_ref, pe_ref, w_ref, xt0_ref, xt1_ref):
    h = x_ref[...] + pos_ref[...] + pe_ref[...]
    xt = jnp.dot(h, w_ref[...], preferred_element_type=jnp.float32)
    xt0_ref[...] = xt[:, :_HD]
    xt1_ref[...] = xt[:, _HD:]


def _prep_a(xr, pos, pe8, w):
    return pl.pallas_call(
        _prepa_body,
        grid=(_PREP_GRID,),
        in_specs=[
            pl.BlockSpec((_PREP_BLK, _D), lambda i: (i, 0)),
            pl.BlockSpec((_PREP_BLK, _D), lambda i: (i, 0)),
            pl.BlockSpec((_PREP_BLK, _D), lambda i: (0, 0)),
            pl.BlockSpec((_D, _D), lambda i: (0, 0)),
        ],
        out_specs=[
            pl.BlockSpec((_PREP_BLK, _HD), lambda i: (i, 0)),
            pl.BlockSpec((_PREP_BLK, _HD), lambda i: (i, 0)),
        ],
        out_shape=[
            jax.ShapeDtypeStruct((_N, _HD), jnp.float32),
            jax.ShapeDtypeStruct((_N, _HD), jnp.float32),
        ],
    )(xr, pos, pe8, w)


_B_NODES = 7168
_B_GRID = _N // _B_NODES
_B_DROWS = _B_NODES // _G
_B_FROWS = _B_NODES // 4


def _prepb_body(xt0_ref, xt1_ref, degp_ref, y0_ref, y1_ref, dinv_ref):
    deg = jnp.sum(degp_ref[...], axis=0) + 1.0
    dinv_n = lax.rsqrt(deg)[:, None]
    y0_ref[...] = xt0_ref[...] * dinv_n
    y1_ref[...] = xt1_ref[...] * dinv_n
    dinv_ref[...] = dinv_n


def _prep_b(xt0, xt1, degp):
    return pl.pallas_call(
        _prepb_body,
        grid=(_B_GRID,),
        in_specs=[
            pl.BlockSpec((_B_NODES, _HD), lambda i: (i, 0)),
            pl.BlockSpec((_B_NODES, _HD), lambda i: (i, 0)),
            pl.BlockSpec((_NC * _NS, _B_NODES), lambda i: (0, i)),
        ],
        out_specs=[
            pl.BlockSpec((_B_NODES, _HD), lambda i: (i, 0)),
            pl.BlockSpec((_B_NODES, _HD), lambda i: (i, 0)),
            pl.BlockSpec((_B_NODES, 1), lambda i: (i, 0)),
        ],
        out_shape=[
            jax.ShapeDtypeStruct((_N, _HD), jnp.float32),
            jax.ShapeDtypeStruct((_N, _HD), jnp.float32),
            jax.ShapeDtypeStruct((_N, 1), jnp.float32),
        ],
    )(xt0, xt1, degp)


_E_GPT = _NGROUPS // _NS
_E_CHUNK = 24
_E_SB = 8
_E_LA = 4


def _fire_gather(c, y0, y1, rowb, rbuf, gsem, j):
    src_idx = rowb.at[pl.ds(j * _G, _G)]

    @pl.when(c == 0)
    def _():
        pltpu.make_async_copy(y0.at[src_idx], rbuf.at[j % _E_SB],
                              gsem).start()

    @pl.when(c == 1)
    def _():
        pltpu.make_async_copy(y1.at[src_idx], rbuf.at[j % _E_SB],
                              gsem).start()


def _edge_body(y0f, y1f, row1d, col1d, a0f, a1f, rowb, colb, rbuf,
               gsem, ssem, a_sh):
    y0, y1, a0_out, a1_out = y0f, y1f, a0f, a1f
    c = lax.axis_index("c")
    s = lax.axis_index("s")
    z16 = jnp.zeros((_L,), jnp.float32)

    @pl.loop(0, _G)
    def _zrow(r):
        rbuf[0, r, pl.ds(0, _L)] = z16
        rbuf[0, r, pl.ds(_L, _L)] = z16

    r0 = s * _ROWS_PER_TILE

    @pl.loop(0, _ROWS_PER_TILE // _G)
    def _zcopy(i):
        pltpu.sync_copy(rbuf.at[0], a_sh.at[pl.ds(r0 + i * _G, _G)])

    plsc.subcore_barrier()

    g0 = s * _E_GPT

    @pl.loop(0, _E_GPT // _E_CHUNK)
    def _chunk(b):
        eb = (g0 + b * _E_CHUNK) * _G
        pltpu.sync_copy(row1d.at[pl.ds(eb, _E_CHUNK * _G)], rowb)
        pltpu.sync_copy(col1d.at[pl.ds(eb, _E_CHUNK * _G)], colb)

        for j in range(_E_LA):
            _fire_gather(c, y0, y1, rowb, rbuf, gsem, j)

        @pl.loop(0, _E_CHUNK)
        def _pipe(j):
            pltpu.make_async_copy(
                y0.at[rowb.at[pl.ds(j * _G, _G)]], rbuf.at[j % _E_SB],
                gsem).wait()
            pltpu.async_copy(
                rbuf.at[j % _E_SB],
                a_sh.at[colb.at[pl.ds(j * _G, _G)]],
                ssem, add=True)

            @pl.when(j >= _E_LA)
            def _():
                jd = j - _E_LA
                pltpu.make_async_copy(
                    rbuf.at[jd % _E_SB],
                    a_sh.at[colb.at[pl.ds(jd * _G, _G)]],
                    ssem).wait()

            @pl.when(j + _E_LA < _E_CHUNK)
            def _():
                _fire_gather(c, y0, y1, rowb, rbuf, gsem, j + _E_LA)

        for j in range(_E_CHUNK - _E_LA, _E_CHUNK):
            pltpu.make_async_copy(
                rbuf.at[j % _E_SB],
                a_sh.at[colb.at[pl.ds(j * _G, _G)]],
                ssem).wait()

    plsc.subcore_barrier()

    @pl.when(c == 0)
    def _():
        pltpu.sync_copy(a_sh.at[pl.ds(r0, _ROWS_PER_TILE)],
                        a0_out.at[pl.ds(r0, _ROWS_PER_TILE)])

    @pl.when(c == 1)
    def _():
        pltpu.sync_copy(a_sh.at[pl.ds(r0, _ROWS_PER_TILE)],
                        a1_out.at[pl.ds(r0, _ROWS_PER_TILE)])


def _edge(y0, y1, row1d, col1d):
    mesh = plsc.VectorSubcoreMesh(core_axis_name="c", subcore_axis_name="s")
    return pl.kernel(
        _edge_body,
        out_type=[
            jax.ShapeDtypeStruct((_N, _HD), jnp.float32),
            jax.ShapeDtypeStruct((_N, _HD), jnp.float32),
        ],
        mesh=mesh,
        scratch_types=[
            pltpu.VMEM((_E_CHUNK * _G,), jnp.int32),
            pltpu.VMEM((_E_CHUNK * _G,), jnp.int32),
            pltpu.VMEM((_E_SB, _G, _HD), jnp.float32),
            pltpu.SemaphoreType.DMA,
            pltpu.SemaphoreType.DMA,
            pltpu.VMEM_SHARED((_N, _HD), jnp.float32),
        ],
        compiler_params=pltpu.CompilerParams(needs_layout_passes=False,
                                             use_tc_tiling_on_sc=False),
    )(y0, y1, row1d, col1d)


_FIN_BLK = 2688
_FIN_GRID = _N // _FIN_BLK


def _fin_body(a0_ref, a1_ref, y0_ref, y1_ref, dinv_ref, batch_ref, b1_ref,
              lw_ref, lb_ref, out_ref, pooled, cnt):
    i = pl.program_id(0)

    @pl.when(i == 0)
    def _():
        pooled[...] = jnp.zeros_like(pooled)
        cnt[...] = jnp.zeros_like(cnt)

    A = jnp.concatenate([a0_ref[...], a1_ref[...]], axis=1)
    y = jnp.concatenate([y0_ref[...], y1_ref[...]], axis=1)
    h = jnp.maximum(dinv_ref[...] * (A + y) + b1_ref[...], 0.0)
    seg = jax.lax.broadcasted_iota(jnp.int32, (_FIN_BLK, _B), 1)
    m = (batch_ref[...] == seg).astype(jnp.float32)
    pooled[...] += lax.dot_general(m, h, (((0,), (0,)), ((), ())),
                                   preferred_element_type=jnp.float32)
    cnt[...] += jnp.sum(m, axis=0, keepdims=True)

    @pl.when(i == _FIN_GRID - 1)
    def _():
        c = jnp.maximum(cnt[...], 1.0)
        pm = pooled[...] / jnp.transpose(c)
        logits = jnp.dot(pm, lw_ref[...],
                         preferred_element_type=jnp.float32) + lb_ref[...]
        z = logits - jnp.max(logits, axis=1, keepdims=True)
        e = jnp.exp(z)
        out_ref[...] = e / jnp.sum(e, axis=1, keepdims=True)


def _finish(a0, a1, y0, y1, dinv, batch2d, b1, lw, lb):
    return pl.pallas_call(
        _fin_body,
        grid=(_FIN_GRID,),
        in_specs=[
            pl.BlockSpec((_FIN_BLK, _HD), lambda i: (i, 0)),
            pl.BlockSpec((_FIN_BLK, _HD), lambda i: (i, 0)),
            pl.BlockSpec((_FIN_BLK, _HD), lambda i: (i, 0)),
            pl.BlockSpec((_FIN_BLK, _HD), lambda i: (i, 0)),
            pl.BlockSpec((_FIN_BLK, 1), lambda i: (i, 0)),
            pl.BlockSpec((_FIN_BLK, 1), lambda i: (i, 0)),
            pl.BlockSpec((1, _D), lambda i: (0, 0)),
            pl.BlockSpec((_D, _NCLS), lambda i: (0, 0)),
            pl.BlockSpec((1, _NCLS), lambda i: (0, 0)),
        ],
        out_specs=pl.BlockSpec((_B, _NCLS), lambda i: (0, 0)),
        out_shape=jax.ShapeDtypeStruct((_B, _NCLS), jnp.float32),
        scratch_shapes=[
            pltpu.VMEM((_B, _D), jnp.float32),
            pltpu.VMEM((1, _B), jnp.float32),
        ],
    )(a0, a1, y0, y1, dinv, batch2d, b1, lw, lb)


def kernel(x, edge_index, batch, batch_size, pos_embed, conv1_w, conv1_b,
           lin_w, lin_b):
    del batch_size
    xr = x.reshape(_N, _D)
    pos = pos_embed.reshape(_N, _D)
    row1d = edge_index[0]
    col1d = edge_index[1]
    pe8 = jnp.asarray(_PE8_NP)

    degp = _hist(col1d)
    xt0, xt1 = _prep_a(xr, pos, pe8, conv1_w)
    y0, y1, dinv = _prep_b(xt0, xt1, degp)
    a0, a1 = _edge(y0, y1, row1d, col1d)
    return _finish(a0, a1, y0, y1, dinv, batch.reshape(_N, 1),
                   conv1_b.reshape(1, _D), lin_w, lin_b.reshape(1, _NCLS))


# speedup vs baseline: 1.0556x; 1.0001x over previous
import functools

import jax
import jax.numpy as jnp
import numpy as np
from jax import lax
from jax.experimental import pallas as pl
from jax.experimental.pallas import tpu as pltpu
from jax.experimental.pallas import tpu_sc as plsc

_B = 128
_NODES_PER = 336
_N = _B * _NODES_PER
_D = 64
_HD = _D // 2
_E = 688128
_NCLS = 10

_NC = 2
_NS = 16
_L = 16

_G = 128
_NGROUPS = _E // _G
_ROWS_PER_TILE = _N // _NS


def _pe_rows(seq_len, d, n=10000.0):
    k = np.arange(seq_len, dtype=np.float32)[:, None]
    i = np.arange(d // 2, dtype=np.float32)[None, :]
    denom = np.power(np.float32(n), 2.0 * i / np.float32(d))
    P = np.zeros((seq_len, d), np.float32)
    P[:, 0::2] = np.sin(k / denom)
    P[:, 1::2] = np.cos(k / denom)
    return P


def _build_pe():
    pe = np.zeros((_NODES_PER, _D), np.float32)
    f4 = _pe_rows(4, _D)
    f16 = _pe_rows(16, _D)
    for i in range(4):
        pe[16 + 16 * i:32 + 16 * i] = f4[i]
    for i in range(16):
        pe[80 + 16 * i:96 + 16 * i] = f16[i]
    return pe


_PE8_NP = np.tile(_build_pe(), (8, 1))


_H_GPT = _NGROUPS // (_NC * _NS)
_H_CHUNK = 24


_H_EPT = _E // (_NC * _NS)
_H_ECHUNK = _H_CHUNK * _G


def _hist_body(col_hbm, deg_out, colb, degl):
    c = lax.axis_index("c")
    s = lax.axis_index("s")
    wid = c * _NS + s
    z16 = jnp.zeros((_L,), jnp.float32)
    ones16 = jnp.ones((_L,), jnp.float32)

    @pl.loop(0, _N // _L)
    def _zero(i):
        degl[pl.ds(i * _L, _L)] = z16

    e0 = wid * _H_EPT

    @pl.loop(0, _H_EPT // _H_ECHUNK)
    def _chunk(b):
        pltpu.sync_copy(col_hbm.at[pl.ds(e0 + b * _H_ECHUNK, _H_ECHUNK)],
                        colb)

        @pl.loop(0, _H_ECHUNK // _L)
        def _vec(k):
            idx = colb[pl.ds(k * _L, _L)]
            plsc.addupdate_scatter(degl, [idx], ones16)

    pltpu.sync_copy(degl, deg_out.at[wid])


def _hist(col1d):
    mesh = plsc.VectorSubcoreMesh(core_axis_name="c", subcore_axis_name="s")
    return pl.kernel(
        _hist_body,
        out_type=jax.ShapeDtypeStruct((_NC * _NS, _N), jnp.float32),
        mesh=mesh,
        scratch_types=[
            pltpu.VMEM((_H_ECHUNK,), jnp.int32),
            pltpu.VMEM((_N,), jnp.float32),
        ],
        compiler_params=pltpu.CompilerParams(needs_layout_passes=False),
    )(col1d)


_PREP_BLK = 2688
_PREP_GRID = _N // _PREP_BLK
_FBLK = _PREP_BLK // 4


def _prepa_body(x_ref, pos_ref, pe_ref, w_ref, xt0_ref, xt1_ref):
    h = x_ref[...] + pos_ref[...] + pe_ref[...]
    xt = jnp.dot(h, w_ref[...], preferred_element_type=jnp.float32)
    xt0_ref[...] = xt[:, :_HD]
    xt1_ref[...] = xt[:, _HD:]


def _prep_a(xr, pos, pe8, w):
    return pl.pallas_call(
        _prepa_body,
        grid=(_PREP_GRID,),
        in_specs=[
            pl.BlockSpec((_PREP_BLK, _D), lambda i: (i, 0)),
            pl.BlockSpec((_PREP_BLK, _D), lambda i: (i, 0)),
            pl.BlockSpec((_PREP_BLK, _D), lambda i: (0, 0)),
            pl.BlockSpec((_D, _D), lambda i: (0, 0)),
        ],
        out_specs=[
            pl.BlockSpec((_PREP_BLK, _HD), lambda i: (i, 0)),
            pl.BlockSpec((_PREP_BLK, _HD), lambda i: (i, 0)),
        ],
        out_shape=[
            jax.ShapeDtypeStruct((_N, _HD), jnp.float32),
            jax.ShapeDtypeStruct((_N, _HD), jnp.float32),
        ],
    )(xr, pos, pe8, w)


_B_NODES = 7168
_B_GRID = _N // _B_NODES
_B_DROWS = _B_NODES // _G
_B_FROWS = _B_NODES // 4


def _prepb_body(xt0_ref, xt1_ref, degp_ref, y0_ref, y1_ref, dinv_ref):
    deg = jnp.sum(degp_ref[...], axis=0) + 1.0
    dinv_n = lax.rsqrt(deg)[:, None]
    y0_ref[...] = xt0_ref[...] * dinv_n
    y1_ref[...] = xt1_ref[...] * dinv_n
    dinv_ref[...] = dinv_n


def _prep_b(xt0, xt1, degp):
    return pl.pallas_call(
        _prepb_body,
        grid=(_B_GRID,),
        in_specs=[
            pl.BlockSpec((_B_NODES, _HD), lambda i: (i, 0)),
            pl.BlockSpec((_B_NODES, _HD), lambda i: (i, 0)),
            pl.BlockSpec((_NC * _NS, _B_NODES), lambda i: (0, i)),
        ],
        out_specs=[
            pl.BlockSpec((_B_NODES, _HD), lambda i: (i, 0)),
            pl.BlockSpec((_B_NODES, _HD), lambda i: (i, 0)),
            pl.BlockSpec((_B_NODES, 1), lambda i: (i, 0)),
        ],
        out_shape=[
            jax.ShapeDtypeStruct((_N, _HD), jnp.float32),
            jax.ShapeDtypeStruct((_N, _HD), jnp.float32),
            jax.ShapeDtypeStruct((_N, 1), jnp.float32),
        ],
    )(xt0, xt1, degp)


_E_GPT = _NGROUPS // _NS
_E_CHUNK = 24
_E_SB = 8
_E_LA = 4


def _fire_gather(c, y0, y1, rowb, rbuf, gsem, j):
    src_idx = rowb.at[pl.ds(j * _G, _G)]

    @pl.when(c == 0)
    def _():
        pltpu.make_async_copy(y0.at[src_idx], rbuf.at[j % _E_SB],
                              gsem).start()

    @pl.when(c == 1)
    def _():
        pltpu.make_async_copy(y1.at[src_idx], rbuf.at[j % _E_SB],
                              gsem).start()


def _edge_body(y0f, y1f, row1d, col1d, a0f, a1f, rowb, colb, rbuf,
               gsem, ssem, a_sh):
    y0, y1, a0_out, a1_out = y0f, y1f, a0f, a1f
    c = lax.axis_index("c")
    s = lax.axis_index("s")
    z16 = jnp.zeros((_L,), jnp.float32)

    @pl.loop(0, _G)
    def _zrow(r):
        rbuf[0, r, pl.ds(0, _L)] = z16
        rbuf[0, r, pl.ds(_L, _L)] = z16

    r0 = s * _ROWS_PER_TILE

    @pl.loop(0, _ROWS_PER_TILE // _G)
    def _zcopy(i):
        pltpu.sync_copy(rbuf.at[0], a_sh.at[pl.ds(r0 + i * _G, _G)])

    plsc.subcore_barrier()

    g0 = s * _E_GPT

    @pl.loop(0, _E_GPT // _E_CHUNK)
    def _chunk(b):
        eb = (g0 + b * _E_CHUNK) * _G
        pltpu.sync_copy(row1d.at[pl.ds(eb, _E_CHUNK * _G)], rowb)
        pltpu.sync_copy(col1d.at[pl.ds(eb, _E_CHUNK * _G)], colb)

        for j in range(_E_LA):
            _fire_gather(c, y0, y1, rowb, rbuf, gsem, j)

        @pl.loop(0, _E_CHUNK, unroll=4)
        def _pipe(j):
            pltpu.make_async_copy(
                y0.at[rowb.at[pl.ds(j * _G, _G)]], rbuf.at[j % _E_SB],
                gsem).wait()
            pltpu.async_copy(
                rbuf.at[j % _E_SB],
                a_sh.at[colb.at[pl.ds(j * _G, _G)]],
                ssem, add=True)

            @pl.when(j >= _E_LA)
            def _():
                jd = j - _E_LA
                pltpu.make_async_copy(
                    rbuf.at[jd % _E_SB],
                    a_sh.at[colb.at[pl.ds(jd * _G, _G)]],
                    ssem).wait()

            @pl.when(j + _E_LA < _E_CHUNK)
            def _():
                _fire_gather(c, y0, y1, rowb, rbuf, gsem, j + _E_LA)

        for j in range(_E_CHUNK - _E_LA, _E_CHUNK):
            pltpu.make_async_copy(
                rbuf.at[j % _E_SB],
                a_sh.at[colb.at[pl.ds(j * _G, _G)]],
                ssem).wait()

    plsc.subcore_barrier()

    @pl.when(c == 0)
    def _():
        pltpu.sync_copy(a_sh.at[pl.ds(r0, _ROWS_PER_TILE)],
                        a0_out.at[pl.ds(r0, _ROWS_PER_TILE)])

    @pl.when(c == 1)
    def _():
        pltpu.sync_copy(a_sh.at[pl.ds(r0, _ROWS_PER_TILE)],
                        a1_out.at[pl.ds(r0, _ROWS_PER_TILE)])


def _edge(y0, y1, row1d, col1d):
    mesh = plsc.VectorSubcoreMesh(core_axis_name="c", subcore_axis_name="s")
    return pl.kernel(
        _edge_body,
        out_type=[
            jax.ShapeDtypeStruct((_N, _HD), jnp.float32),
            jax.ShapeDtypeStruct((_N, _HD), jnp.float32),
        ],
        mesh=mesh,
        scratch_types=[
            pltpu.VMEM((_E_CHUNK * _G,), jnp.int32),
            pltpu.VMEM((_E_CHUNK * _G,), jnp.int32),
            pltpu.VMEM((_E_SB, _G, _HD), jnp.float32),
            pltpu.SemaphoreType.DMA,
            pltpu.SemaphoreType.DMA,
            pltpu.VMEM_SHARED((_N, _HD), jnp.float32),
        ],
        compiler_params=pltpu.CompilerParams(needs_layout_passes=False,
                                             use_tc_tiling_on_sc=False),
    )(y0, y1, row1d, col1d)


_FIN_BLK = 2688
_FIN_GRID = _N // _FIN_BLK


def _fin_body(a0_ref, a1_ref, y0_ref, y1_ref, dinv_ref, batch_ref, b1_ref,
              lw_ref, lb_ref, out_ref, pooled, cnt):
    i = pl.program_id(0)

    @pl.when(i == 0)
    def _():
        pooled[...] = jnp.zeros_like(pooled)
        cnt[...] = jnp.zeros_like(cnt)

    A = jnp.concatenate([a0_ref[...], a1_ref[...]], axis=1)
    y = jnp.concatenate([y0_ref[...], y1_ref[...]], axis=1)
    h = jnp.maximum(dinv_ref[...] * (A + y) + b1_ref[...], 0.0)
    seg = jax.lax.broadcasted_iota(jnp.int32, (_FIN_BLK, _B), 1)
    m = (batch_ref[...] == seg).astype(jnp.float32)
    pooled[...] += lax.dot_general(m, h, (((0,), (0,)), ((), ())),
                                   preferred_element_type=jnp.float32)
    cnt[...] += jnp.sum(m, axis=0, keepdims=True)

    @pl.when(i == _FIN_GRID - 1)
    def _():
        c = jnp.maximum(cnt[...], 1.0)
        pm = pooled[...] / jnp.transpose(c)
        logits = jnp.dot(pm, lw_ref[...],
                         preferred_element_type=jnp.float32) + lb_ref[...]
        z = logits - jnp.max(logits, axis=1, keepdims=True)
        e = jnp.exp(z)
        out_ref[...] = e / jnp.sum(e, axis=1, keepdims=True)


def _finish(a0, a1, y0, y1, dinv, batch2d, b1, lw, lb):
    return pl.pallas_call(
        _fin_body,
        grid=(_FIN_GRID,),
        in_specs=[
            pl.BlockSpec((_FIN_BLK, _HD), lambda i: (i, 0)),
            pl.BlockSpec((_FIN_BLK, _HD), lambda i: (i, 0)),
            pl.BlockSpec((_FIN_BLK, _HD), lambda i: (i, 0)),
            pl.BlockSpec((_FIN_BLK, _HD), lambda i: (i, 0)),
            pl.BlockSpec((_FIN_BLK, 1), lambda i: (i, 0)),
            pl.BlockSpec((_FIN_BLK, 1), lambda i: (i, 0)),
            pl.BlockSpec((1, _D), lambda i: (0, 0)),
            pl.BlockSpec((_D, _NCLS), lambda i: (0, 0)),
            pl.BlockSpec((1, _NCLS), lambda i: (0, 0)),
        ],
        out_specs=pl.BlockSpec((_B, _NCLS), lambda i: (0, 0)),
        out_shape=jax.ShapeDtypeStruct((_B, _NCLS), jnp.float32),
        scratch_shapes=[
            pltpu.VMEM((_B, _D), jnp.float32),
            pltpu.VMEM((1, _B), jnp.float32),
        ],
    )(a0, a1, y0, y1, dinv, batch2d, b1, lw, lb)


def kernel(x, edge_index, batch, batch_size, pos_embed, conv1_w, conv1_b,
           lin_w, lin_b):
    del batch_size
    xr = x.reshape(_N, _D)
    pos = pos_embed.reshape(_N, _D)
    row1d = edge_index[0]
    col1d = edge_index[1]
    pe8 = jnp.asarray(_PE8_NP)

    degp = _hist(col1d)
    xt0, xt1 = _prep_a(xr, pos, pe8, conv1_w)
    y0, y1, dinv = _prep_b(xt0, xt1, degp)
    a0, a1 = _edge(y0, y1, row1d, col1d)
    return _finish(a0, a1, y0, y1, dinv, batch.reshape(_N, 1),
                   conv1_b.reshape(1, _D), lin_w, lin_b.reshape(1, _NCLS))
